# baseline (speedup 1.0000x reference)
import functools

import jax
import jax.numpy as jnp
from jax import lax
from jax.experimental import pallas as pl
from jax.experimental.pallas import tpu as pltpu
from jax.experimental.pallas import tpu_sc as plsc

_B = 16384
_N_UP = 1000000
_D_UP = 64
_D_SM = 32
_D_MODEL = 512

_NC = 2
_NS = 16
_NW = _NC * _NS
_BPW = _B // _NW
_CHUNK = 128
_NCHUNK = _BPW // _CHUNK

_RB = 2048


def _repack_body(tblT, out):
    tb = jnp.transpose(tblT[...].astype(jnp.bfloat16))
    t16 = lax.bitcast_convert_type(tb, jnp.uint16).astype(jnp.uint32)
    t4 = t16.reshape(_RB // 2, 2, _D_UP)
    packed = t4[:, 0, :] | (t4[:, 1, :] << 16)
    p4 = packed.reshape(_RB // 4, 2, _D_UP)
    out_u = jnp.concatenate([p4[:, 0, :], p4[:, 1, :]], axis=-1)
    out[...] = lax.bitcast_convert_type(out_u, jnp.float32)


def _repack(tblT):
    grid = (pl.cdiv(_N_UP, _RB),)
    return pl.pallas_call(
        _repack_body,
        grid=grid,
        in_specs=[pl.BlockSpec((_D_UP, _RB), lambda i: (0, i))],
        out_specs=pl.BlockSpec((_RB // 4, 2 * _D_UP), lambda i: (i, 0)),
        out_shape=jax.ShapeDtypeStruct((_N_UP // 4, 2 * _D_UP), jnp.float32),
        compiler_params=pltpu.CompilerParams(
            dimension_semantics=("parallel",),
        ),
    )(tblT)


def _sc_gather_body(jh, R, out, idx_v, rows_v, sem):
    wid = lax.axis_index("s") * _NC + lax.axis_index("c")
    base = wid * _BPW

    pltpu.sync_copy(jh.at[pl.ds(base, _BPW)], idx_v)
    copies = []
    for j in range(_NCHUNK):
        copies.append(pltpu.async_copy(
            R.at[idx_v.at[pl.ds(j * _CHUNK, _CHUNK)]],
            rows_v.at[pl.ds(j * _CHUNK, _CHUNK)], sem))
    for c in copies:
        c.wait()
    pltpu.sync_copy(rows_v, out.at[pl.ds(base, _BPW)])


@functools.cache
def _sc_gather():
    return pl.kernel(
        _sc_gather_body,
        out_type=jax.ShapeDtypeStruct((_B, 2 * _D_UP), jnp.float32),
        mesh=plsc.VectorSubcoreMesh(core_axis_name="c", subcore_axis_name="s",
                                    num_cores=_NC, num_subcores=_NS),
        scratch_types=[
            pltpu.VMEM((_BPW,), jnp.int32),
            pltpu.VMEM((_BPW, 2 * _D_UP), jnp.float32),
            pltpu.SemaphoreType.DMA,
        ],
    )


_BLK = 2048


def _onehot_part(idx_ref, tab_ref, w_slice, n_rows):
    idx = idx_ref[0]
    iota = lax.broadcasted_iota(jnp.int32, (n_rows, _BLK), 0)
    oh = jnp.equal(iota, idx).astype(jnp.float32)
    m = jnp.dot(tab_ref[...], w_slice, preferred_element_type=jnp.float32)
    return lax.dot_general(oh, m, (((0,), (0,)), ((), ())),
                           preferred_element_type=jnp.float32)


def _mlp_body(g_up, rem, pf, dt, lk, pf_tab, dt_tab, lk_tab,
              w1dd, W1, b1, gamma, beta, W2, b2, out):
    g_u = lax.bitcast_convert_type(g_up[...], jnp.uint32)
    f_lo = lax.bitcast_convert_type(g_u << 16, jnp.float32)
    f_hi = lax.bitcast_convert_type(g_u & jnp.uint32(0xFFFF0000), jnp.float32)
    u2 = lax.broadcasted_iota(jnp.int32, (_BLK, 2 * _D_UP), 1) // _D_UP
    m_lo = jnp.equal(rem[...], 2 * u2).astype(jnp.float32)
    m_hi = jnp.equal(rem[...], 2 * u2 + 1).astype(jnp.float32)
    e = f_lo * m_lo + f_hi * m_hi
    h = jnp.dot(e, w1dd[...], preferred_element_type=jnp.float32)
    h = h + _onehot_part(pf, pf_tab, W1[64:96, :], 10)
    h = h + _onehot_part(dt, dt_tab, W1[96:128, :], 12)
    h = h + _onehot_part(lk, lk_tab, W1[128:160, :], 10)
    h = h + b1[...]
    mean = jnp.mean(h, axis=-1, keepdims=True)
    c = h - mean
    var = jnp.mean(c * c, axis=-1, keepdims=True)
    h = c * lax.rsqrt(var + 1e-5) * gamma[...] + beta[...]
    h = jnp.maximum(h, 0.0)
    out[...] = jnp.dot(h, W2[...], preferred_element_type=jnp.float32) + b2[...]


def _mlp(g_up, rem, pf, dt, lk, pf_tab, dt_tab, lk_tab,
         w1dd, W1, b1, gamma, beta, W2, b2):
    grid = (_B // _BLK,)
    return pl.pallas_call(
        _mlp_body,
        grid=grid,
        in_specs=[
            pl.BlockSpec((_BLK, 2 * _D_UP), lambda i: (i, 0)),
            pl.BlockSpec((_BLK, 1), lambda i: (i, 0)),
            pl.BlockSpec((1, 1, _BLK), lambda i: (i, 0, 0)),
            pl.BlockSpec((1, 1, _BLK), lambda i: (i, 0, 0)),
            pl.BlockSpec((1, 1, _BLK), lambda i: (i, 0, 0)),
            pl.BlockSpec((10, _D_SM), lambda i: (0, 0)),
            pl.BlockSpec((12, _D_SM), lambda i: (0, 0)),
            pl.BlockSpec((10, _D_SM), lambda i: (0, 0)),
            pl.BlockSpec((2 * _D_UP, _D_MODEL), lambda i: (0, 0)),
            pl.BlockSpec((160, _D_MODEL), lambda i: (0, 0)),
            pl.BlockSpec((1, _D_MODEL), lambda i: (0, 0)),
            pl.BlockSpec((1, _D_MODEL), lambda i: (0, 0)),
            pl.BlockSpec((1, _D_MODEL), lambda i: (0, 0)),
            pl.BlockSpec((_D_MODEL, _D_MODEL), lambda i: (0, 0)),
            pl.BlockSpec((1, _D_MODEL), lambda i: (0, 0)),
        ],
        out_specs=pl.BlockSpec((_BLK, _D_MODEL), lambda i: (i, 0)),
        out_shape=jax.ShapeDtypeStruct((_B, _D_MODEL), jnp.float32),
        compiler_params=pltpu.CompilerParams(
            dimension_semantics=("arbitrary",),
        ),
    )(g_up, rem, pf, dt, lk, pf_tab, dt_tab, lk_tab,
      w1dd, W1, b1, gamma, beta, W2, b2)


def kernel(uploader, platform, date, likes, uploader_table, platform_table,
           date_table, likes_table, W1, b1, gamma, beta, W2, b2):
    nb = _B // _BLK
    upl = uploader.astype(jnp.int32)
    R = _repack(uploader_table.T)
    g_up = _sc_gather()(upl >> 2, R)

    w1dd = jnp.tile(W1[0:_D_UP, :], (2, 1))
    out = _mlp(g_up, (upl & 3).reshape(_B, 1),
               platform.astype(jnp.int32).reshape(nb, 1, _BLK),
               date.astype(jnp.int32).reshape(nb, 1, _BLK),
               likes.astype(jnp.int32).reshape(nb, 1, _BLK),
               platform_table, date_table, likes_table,
               w1dd,
               W1, b1.reshape(1, _D_MODEL), gamma.reshape(1, _D_MODEL),
               beta.reshape(1, _D_MODEL), W2, b2.reshape(1, _D_MODEL))
    return out[:, None, :]


# speedup vs baseline: 1.1858x; 1.1858x over previous
import functools

import jax
import jax.numpy as jnp
from jax import lax
from jax.experimental import pallas as pl
from jax.experimental.pallas import tpu as pltpu
from jax.experimental.pallas import tpu_sc as plsc

_B = 16384
_N_UP = 1000000
_D_UP = 64
_D_SM = 32
_D_MODEL = 512

_NC = 2
_NS = 16
_NW = _NC * _NS
_BPW = _B // _NW
_CHUNK = 128
_NCHUNK = _BPW // _CHUNK

_RB = 2048


def _repack_body(tblT, out):
    t = jnp.transpose(tblT[...])
    out[...] = jnp.concatenate([t[: _RB // 2, :], t[_RB // 2:, :]], axis=-1)


def _repack(tblT):
    grid = (pl.cdiv(_N_UP, _RB),)
    return pl.pallas_call(
        _repack_body,
        grid=grid,
        in_specs=[pl.BlockSpec((_D_UP, _RB), lambda i: (0, i))],
        out_specs=pl.BlockSpec((_RB // 2, 2 * _D_UP), lambda i: (i, 0)),
        out_shape=jax.ShapeDtypeStruct(
            (pl.cdiv(_N_UP, _RB) * (_RB // 2), 2 * _D_UP), jnp.float32),
        compiler_params=pltpu.CompilerParams(
            dimension_semantics=("parallel",),
        ),
    )(tblT)


def _sc_gather_body(jh, R, out, idx_v, rows_v, sem):
    wid = lax.axis_index("s") * _NC + lax.axis_index("c")
    base = wid * _BPW

    pltpu.sync_copy(jh.at[pl.ds(base, _BPW)], idx_v)
    copies = []
    for j in range(_NCHUNK):
        copies.append(pltpu.async_copy(
            R.at[idx_v.at[pl.ds(j * _CHUNK, _CHUNK)]],
            rows_v.at[pl.ds(j * _CHUNK, _CHUNK)], sem))
    for c in copies:
        c.wait()
    pltpu.sync_copy(rows_v, out.at[pl.ds(base, _BPW)])


@functools.cache
def _sc_gather():
    return pl.kernel(
        _sc_gather_body,
        out_type=jax.ShapeDtypeStruct((_B, 2 * _D_UP), jnp.float32),
        mesh=plsc.VectorSubcoreMesh(core_axis_name="c", subcore_axis_name="s",
                                    num_cores=_NC, num_subcores=_NS),
        scratch_types=[
            pltpu.VMEM((_BPW,), jnp.int32),
            pltpu.VMEM((_BPW, 2 * _D_UP), jnp.float32),
            pltpu.SemaphoreType.DMA,
        ],
    )


_BLK = 2048


def _onehot_part(idx_ref, tab_ref, w_slice, n_rows):
    idx = idx_ref[0]
    iota = lax.broadcasted_iota(jnp.int32, (n_rows, _BLK), 0)
    oh = jnp.equal(iota, idx).astype(jnp.float32)
    m = jnp.dot(tab_ref[...], w_slice, preferred_element_type=jnp.float32)
    return lax.dot_general(oh, m, (((0,), (0,)), ((), ())),
                           preferred_element_type=jnp.float32)


def _mlp_body(g_up, rem, pf, dt, lk, pf_tab, dt_tab, lk_tab,
              w1dd, W1, b1, gamma, beta, W2, b2, out):
    half = lax.broadcasted_iota(jnp.int32, (_BLK, 2 * _D_UP), 1) // _D_UP
    m = jnp.equal(rem[...], half).astype(jnp.float32)
    h = jnp.dot(g_up[...] * m, w1dd[...], preferred_element_type=jnp.float32)
    h = h + _onehot_part(pf, pf_tab, W1[64:96, :], 10)
    h = h + _onehot_part(dt, dt_tab, W1[96:128, :], 12)
    h = h + _onehot_part(lk, lk_tab, W1[128:160, :], 10)
    h = h + b1[...]
    mean = jnp.mean(h, axis=-1, keepdims=True)
    c = h - mean
    var = jnp.mean(c * c, axis=-1, keepdims=True)
    h = c * lax.rsqrt(var + 1e-5) * gamma[...] + beta[...]
    h = jnp.maximum(h, 0.0)
    out[...] = jnp.dot(h, W2[...], preferred_element_type=jnp.float32) + b2[...]


def _mlp(g_up, rem, pf, dt, lk, pf_tab, dt_tab, lk_tab,
         w1dd, W1, b1, gamma, beta, W2, b2):
    grid = (_B // _BLK,)
    return pl.pallas_call(
        _mlp_body,
        grid=grid,
        in_specs=[
            pl.BlockSpec((_BLK, 2 * _D_UP), lambda i: (i, 0)),
            pl.BlockSpec((_BLK, 1), lambda i: (i, 0)),
            pl.BlockSpec((1, 1, _BLK), lambda i: (i, 0, 0)),
            pl.BlockSpec((1, 1, _BLK), lambda i: (i, 0, 0)),
            pl.BlockSpec((1, 1, _BLK), lambda i: (i, 0, 0)),
            pl.BlockSpec((10, _D_SM), lambda i: (0, 0)),
            pl.BlockSpec((12, _D_SM), lambda i: (0, 0)),
            pl.BlockSpec((10, _D_SM), lambda i: (0, 0)),
            pl.BlockSpec((2 * _D_UP, _D_MODEL), lambda i: (0, 0)),
            pl.BlockSpec((160, _D_MODEL), lambda i: (0, 0)),
            pl.BlockSpec((1, _D_MODEL), lambda i: (0, 0)),
            pl.BlockSpec((1, _D_MODEL), lambda i: (0, 0)),
            pl.BlockSpec((1, _D_MODEL), lambda i: (0, 0)),
            pl.BlockSpec((_D_MODEL, _D_MODEL), lambda i: (0, 0)),
            pl.BlockSpec((1, _D_MODEL), lambda i: (0, 0)),
        ],
        out_specs=pl.BlockSpec((_BLK, _D_MODEL), lambda i: (i, 0)),
        out_shape=jax.ShapeDtypeStruct((_B, _D_MODEL), jnp.float32),
        compiler_params=pltpu.CompilerParams(
            dimension_semantics=("arbitrary",),
        ),
    )(g_up, rem, pf, dt, lk, pf_tab, dt_tab, lk_tab,
      w1dd, W1, b1, gamma, beta, W2, b2)


def kernel(uploader, platform, date, likes, uploader_table, platform_table,
           date_table, likes_table, W1, b1, gamma, beta, W2, b2):
    nb = _B // _BLK
    upl = uploader.astype(jnp.int32)
    R = _repack(uploader_table.T)
    w = upl % _RB
    q = (upl // _RB) * (_RB // 2) + (w % (_RB // 2))
    sel = w // (_RB // 2)
    g_up = _sc_gather()(q, R)

    w1dd = jnp.tile(W1[0:_D_UP, :], (2, 1))
    out = _mlp(g_up, sel.reshape(_B, 1),
               platform.astype(jnp.int32).reshape(nb, 1, _BLK),
               date.astype(jnp.int32).reshape(nb, 1, _BLK),
               likes.astype(jnp.int32).reshape(nb, 1, _BLK),
               platform_table, date_table, likes_table,
               w1dd,
               W1, b1.reshape(1, _D_MODEL), gamma.reshape(1, _D_MODEL),
               beta.reshape(1, _D_MODEL), W2, b2.reshape(1, _D_MODEL))
    return out[:, None, :]


# speedup vs baseline: 1.3505x; 1.1390x over previous
import functools

import jax
import jax.numpy as jnp
from jax import lax
from jax.experimental import pallas as pl
from jax.experimental.pallas import tpu as pltpu
from jax.experimental.pallas import tpu_sc as plsc

_B = 16384
_N_UP = 1000000
_D_UP = 64
_D_SM = 32
_D_MODEL = 512

_NC = 2
_NS = 16
_NW = _NC * _NS
_BPW = _B // _NW
_CHUNK = 128
_NCHUNK = _BPW // _CHUNK

_RB = 2048


def _repack_body(tblT, out):
    tb = jnp.transpose(tblT[...].astype(jnp.bfloat16))
    t16 = lax.bitcast_convert_type(tb, jnp.uint16).astype(jnp.uint32)
    p = t16[: _RB // 2, :] | (t16[_RB // 2:, :] << 16)
    out_u = jnp.concatenate([p[: _RB // 4, :], p[_RB // 4:, :]], axis=-1)
    out[...] = lax.bitcast_convert_type(out_u, jnp.float32)


def _repack(tblT):
    grid = (pl.cdiv(_N_UP, _RB),)
    return pl.pallas_call(
        _repack_body,
        grid=grid,
        in_specs=[pl.BlockSpec((_D_UP, _RB), lambda i: (0, i))],
        out_specs=pl.BlockSpec((_RB // 4, 2 * _D_UP), lambda i: (i, 0)),
        out_shape=jax.ShapeDtypeStruct(
            (pl.cdiv(_N_UP, _RB) * (_RB // 4), 2 * _D_UP), jnp.float32),
        compiler_params=pltpu.CompilerParams(
            dimension_semantics=("parallel",),
        ),
    )(tblT)


def _sc_gather_body(jh, R, out, idx_v, rows_v, sem):
    wid = lax.axis_index("s") * _NC + lax.axis_index("c")
    base = wid * _BPW

    pltpu.sync_copy(jh.at[pl.ds(base, _BPW)], idx_v)
    copies = []
    for j in range(_NCHUNK):
        copies.append(pltpu.async_copy(
            R.at[idx_v.at[pl.ds(j * _CHUNK, _CHUNK)]],
            rows_v.at[pl.ds(j * _CHUNK, _CHUNK)], sem))
    for c in copies:
        c.wait()
    pltpu.sync_copy(rows_v, out.at[pl.ds(base, _BPW)])


@functools.cache
def _sc_gather():
    return pl.kernel(
        _sc_gather_body,
        out_type=jax.ShapeDtypeStruct((_B, 2 * _D_UP), jnp.float32),
        mesh=plsc.VectorSubcoreMesh(core_axis_name="c", subcore_axis_name="s",
                                    num_cores=_NC, num_subcores=_NS),
        scratch_types=[
            pltpu.VMEM((_BPW,), jnp.int32),
            pltpu.VMEM((_BPW, 2 * _D_UP), jnp.float32),
            pltpu.SemaphoreType.DMA,
        ],
    )


_BLK = 2048


def _onehot_part(idx_ref, tab_ref, w_slice, n_rows):
    idx = idx_ref[0]
    iota = lax.broadcasted_iota(jnp.int32, (n_rows, _BLK), 0)
    oh = jnp.equal(iota, idx).astype(jnp.float32)
    m = jnp.dot(tab_ref[...], w_slice, preferred_element_type=jnp.float32)
    return lax.dot_general(oh, m, (((0,), (0,)), ((), ())),
                           preferred_element_type=jnp.float32)


def _mlp_body(g_up, rem, pf, dt, lk, pf_tab, dt_tab, lk_tab,
              w1dd, W1, b1, gamma, beta, W2, b2, out):
    g_u = lax.bitcast_convert_type(g_up[...], jnp.uint32)
    f_lo = lax.bitcast_convert_type(g_u << 16, jnp.float32)
    f_hi = lax.bitcast_convert_type(g_u & jnp.uint32(0xFFFF0000), jnp.float32)
    u2 = lax.broadcasted_iota(jnp.int32, (_BLK, 2 * _D_UP), 1) // _D_UP
    m_lo = jnp.equal(rem[...], 2 * u2).astype(jnp.float32)
    m_hi = jnp.equal(rem[...], 2 * u2 + 1).astype(jnp.float32)
    e = f_lo * m_lo + f_hi * m_hi
    h = jnp.dot(e, w1dd[...], preferred_element_type=jnp.float32)
    h = h + _onehot_part(pf, pf_tab, W1[64:96, :], 10)
    h = h + _onehot_part(dt, dt_tab, W1[96:128, :], 12)
    h = h + _onehot_part(lk, lk_tab, W1[128:160, :], 10)
    h = h + b1[...]
    mean = jnp.mean(h, axis=-1, keepdims=True)
    c = h - mean
    var = jnp.mean(c * c, axis=-1, keepdims=True)
    h = c * lax.rsqrt(var + 1e-5) * gamma[...] + beta[...]
    h = jnp.maximum(h, 0.0)
    out[...] = jnp.dot(h, W2[...], preferred_element_type=jnp.float32) + b2[...]


def _mlp(g_up, rem, pf, dt, lk, pf_tab, dt_tab, lk_tab,
         w1dd, W1, b1, gamma, beta, W2, b2):
    grid = (_B // _BLK,)
    return pl.pallas_call(
        _mlp_body,
        grid=grid,
        in_specs=[
            pl.BlockSpec((_BLK, 2 * _D_UP), lambda i: (i, 0)),
            pl.BlockSpec((_BLK, 1), lambda i: (i, 0)),
            pl.BlockSpec((1, 1, _BLK), lambda i: (i, 0, 0)),
            pl.BlockSpec((1, 1, _BLK), lambda i: (i, 0, 0)),
            pl.BlockSpec((1, 1, _BLK), lambda i: (i, 0, 0)),
            pl.BlockSpec((10, _D_SM), lambda i: (0, 0)),
            pl.BlockSpec((12, _D_SM), lambda i: (0, 0)),
            pl.BlockSpec((10, _D_SM), lambda i: (0, 0)),
            pl.BlockSpec((2 * _D_UP, _D_MODEL), lambda i: (0, 0)),
            pl.BlockSpec((160, _D_MODEL), lambda i: (0, 0)),
            pl.BlockSpec((1, _D_MODEL), lambda i: (0, 0)),
            pl.BlockSpec((1, _D_MODEL), lambda i: (0, 0)),
            pl.BlockSpec((1, _D_MODEL), lambda i: (0, 0)),
            pl.BlockSpec((_D_MODEL, _D_MODEL), lambda i: (0, 0)),
            pl.BlockSpec((1, _D_MODEL), lambda i: (0, 0)),
        ],
        out_specs=pl.BlockSpec((_BLK, _D_MODEL), lambda i: (i, 0)),
        out_shape=jax.ShapeDtypeStruct((_B, _D_MODEL), jnp.float32),
        compiler_params=pltpu.CompilerParams(
            dimension_semantics=("arbitrary",),
        ),
    )(g_up, rem, pf, dt, lk, pf_tab, dt_tab, lk_tab,
      w1dd, W1, b1, gamma, beta, W2, b2)


def kernel(uploader, platform, date, likes, uploader_table, platform_table,
           date_table, likes_table, W1, b1, gamma, beta, W2, b2):
    nb = _B // _BLK
    upl = uploader.astype(jnp.int32)
    R = _repack(uploader_table.T)
    w = upl % _RB
    eta = w // (_RB // 2)
    r = w % (_RB // 2)
    v = r // (_RB // 4)
    q = (upl // _RB) * (_RB // 4) + (r % (_RB // 4))
    rem = 2 * v + eta
    g_up = _sc_gather()(q, R)

    w1dd = jnp.tile(W1[0:_D_UP, :], (2, 1))
    out = _mlp(g_up, rem.reshape(_B, 1),
               platform.astype(jnp.int32).reshape(nb, 1, _BLK),
               date.astype(jnp.int32).reshape(nb, 1, _BLK),
               likes.astype(jnp.int32).reshape(nb, 1, _BLK),
               platform_table, date_table, likes_table,
               w1dd,
               W1, b1.reshape(1, _D_MODEL), gamma.reshape(1, _D_MODEL),
               beta.reshape(1, _D_MODEL), W2, b2.reshape(1, _D_MODEL))
    return out[:, None, :]


# speedup vs baseline: 1.8568x; 1.3749x over previous
import functools

import jax
import jax.numpy as jnp
from jax import lax
from jax.experimental import pallas as pl
from jax.experimental.pallas import tpu as pltpu
from jax.experimental.pallas import tpu_sc as plsc

_B = 16384
_N_UP = 1000000
_D_UP = 64
_D_SM = 32
_D_MODEL = 512

_NC = 2
_NS = 16
_NW = _NC * _NS
_BPW = _B // _NW
_CHUNK = 128
_NCHUNK = _BPW // _CHUNK

_RB = 4096


def _repack_body(tblT, out):
    tb = jnp.transpose(tblT[...].astype(jnp.bfloat16))
    t16 = lax.bitcast_convert_type(tb, jnp.uint16).astype(jnp.uint32)
    p = t16[: _RB // 2, :] | (t16[_RB // 2:, :] << 16)
    out_u = jnp.concatenate([p[: _RB // 4, :], p[_RB // 4:, :]], axis=-1)
    out[...] = lax.bitcast_convert_type(out_u, jnp.float32)


def _repack(tblT):
    grid = (pl.cdiv(_N_UP, _RB),)
    return pl.pallas_call(
        _repack_body,
        grid=grid,
        in_specs=[pl.BlockSpec((_D_UP, _RB), lambda i: (0, i))],
        out_specs=pl.BlockSpec((_RB // 4, 2 * _D_UP), lambda i: (i, 0)),
        out_shape=jax.ShapeDtypeStruct(
            (pl.cdiv(_N_UP, _RB) * (_RB // 4), 2 * _D_UP), jnp.float32),
        compiler_params=pltpu.CompilerParams(
            dimension_semantics=("parallel",),
        ),
    )(tblT)


def _sc_gather_body(jh, R, out, idx_v, rows_v, sem):
    wid = lax.axis_index("s") * _NC + lax.axis_index("c")
    base = wid * _BPW

    pltpu.sync_copy(jh.at[pl.ds(base, _BPW)], idx_v)
    copies = []
    for j in range(_NCHUNK):
        copies.append(pltpu.async_copy(
            R.at[idx_v.at[pl.ds(j * _CHUNK, _CHUNK)]],
            rows_v.at[pl.ds(j * _CHUNK, _CHUNK)], sem))
    for c in copies:
        c.wait()
    pltpu.sync_copy(rows_v, out.at[pl.ds(base, _BPW)])


@functools.cache
def _sc_gather():
    return pl.kernel(
        _sc_gather_body,
        out_type=jax.ShapeDtypeStruct((_B, 2 * _D_UP), jnp.float32),
        mesh=plsc.VectorSubcoreMesh(core_axis_name="c", subcore_axis_name="s",
                                    num_cores=_NC, num_subcores=_NS),
        scratch_types=[
            pltpu.VMEM((_BPW,), jnp.int32),
            pltpu.VMEM((_BPW, 2 * _D_UP), jnp.float32),
            pltpu.SemaphoreType.DMA,
        ],
    )


_BLK = 2048


def _onehot_part(idx_ref, tab_ref, w_slice, n_rows):
    idx = idx_ref[0]
    iota = lax.broadcasted_iota(jnp.int32, (n_rows, _BLK), 0)
    oh = jnp.equal(iota, idx).astype(jnp.float32)
    m = jnp.dot(tab_ref[...], w_slice, preferred_element_type=jnp.float32)
    return lax.dot_general(oh, m, (((0,), (0,)), ((), ())),
                           preferred_element_type=jnp.float32)


def _mlp_body(g_up, rem, pf, dt, lk, pf_tab, dt_tab, lk_tab,
              w1dd, W1, b1, gamma, beta, W2, b2, out):
    g_u = lax.bitcast_convert_type(g_up[...], jnp.uint32)
    f_lo = lax.bitcast_convert_type(g_u << 16, jnp.float32)
    f_hi = lax.bitcast_convert_type(g_u & jnp.uint32(0xFFFF0000), jnp.float32)
    u2 = lax.broadcasted_iota(jnp.int32, (_BLK, 2 * _D_UP), 1) // _D_UP
    m_lo = jnp.equal(rem[...], 2 * u2).astype(jnp.float32)
    m_hi = jnp.equal(rem[...], 2 * u2 + 1).astype(jnp.float32)
    e = f_lo * m_lo + f_hi * m_hi
    h = jnp.dot(e, w1dd[...], preferred_element_type=jnp.float32)
    h = h + _onehot_part(pf, pf_tab, W1[64:96, :], 10)
    h = h + _onehot_part(dt, dt_tab, W1[96:128, :], 12)
    h = h + _onehot_part(lk, lk_tab, W1[128:160, :], 10)
    h = h + b1[...]
    mean = jnp.mean(h, axis=-1, keepdims=True)
    c = h - mean
    var = jnp.mean(c * c, axis=-1, keepdims=True)
    h = c * lax.rsqrt(var + 1e-5) * gamma[...] + beta[...]
    h = jnp.maximum(h, 0.0)
    out[...] = jnp.dot(h, W2[...], preferred_element_type=jnp.float32) + b2[...]


def _mlp(g_up, rem, pf, dt, lk, pf_tab, dt_tab, lk_tab,
         w1dd, W1, b1, gamma, beta, W2, b2):
    grid = (_B // _BLK,)
    return pl.pallas_call(
        _mlp_body,
        grid=grid,
        in_specs=[
            pl.BlockSpec((_BLK, 2 * _D_UP), lambda i: (i, 0)),
            pl.BlockSpec((_BLK, 1), lambda i: (i, 0)),
            pl.BlockSpec((1, 1, _BLK), lambda i: (i, 0, 0)),
            pl.BlockSpec((1, 1, _BLK), lambda i: (i, 0, 0)),
            pl.BlockSpec((1, 1, _BLK), lambda i: (i, 0, 0)),
            pl.BlockSpec((10, _D_SM), lambda i: (0, 0)),
            pl.BlockSpec((12, _D_SM), lambda i: (0, 0)),
            pl.BlockSpec((10, _D_SM), lambda i: (0, 0)),
            pl.BlockSpec((2 * _D_UP, _D_MODEL), lambda i: (0, 0)),
            pl.BlockSpec((160, _D_MODEL), lambda i: (0, 0)),
            pl.BlockSpec((1, _D_MODEL), lambda i: (0, 0)),
            pl.BlockSpec((1, _D_MODEL), lambda i: (0, 0)),
            pl.BlockSpec((1, _D_MODEL), lambda i: (0, 0)),
            pl.BlockSpec((_D_MODEL, _D_MODEL), lambda i: (0, 0)),
            pl.BlockSpec((1, _D_MODEL), lambda i: (0, 0)),
        ],
        out_specs=pl.BlockSpec((_BLK, _D_MODEL), lambda i: (i, 0)),
        out_shape=jax.ShapeDtypeStruct((_B, _D_MODEL), jnp.float32),
        compiler_params=pltpu.CompilerParams(
            dimension_semantics=("arbitrary",),
        ),
    )(g_up, rem, pf, dt, lk, pf_tab, dt_tab, lk_tab,
      w1dd, W1, b1, gamma, beta, W2, b2)


def kernel(uploader, platform, date, likes, uploader_table, platform_table,
           date_table, likes_table, W1, b1, gamma, beta, W2, b2):
    nb = _B // _BLK
    upl = uploader.astype(jnp.int32)
    R = _repack(uploader_table.T)
    w = upl % _RB
    eta = w // (_RB // 2)
    r = w % (_RB // 2)
    v = r // (_RB // 4)
    q = (upl // _RB) * (_RB // 4) + (r % (_RB // 4))
    rem = 2 * v + eta
    g_up = _sc_gather()(q, R)

    w1dd = jnp.tile(W1[0:_D_UP, :], (2, 1))
    out = _mlp(g_up, rem.reshape(_B, 1),
               platform.astype(jnp.int32).reshape(nb, 1, _BLK),
               date.astype(jnp.int32).reshape(nb, 1, _BLK),
               likes.astype(jnp.int32).reshape(nb, 1, _BLK),
               platform_table, date_table, likes_table,
               w1dd,
               W1, b1.reshape(1, _D_MODEL), gamma.reshape(1, _D_MODEL),
               beta.reshape(1, _D_MODEL), W2, b2.reshape(1, _D_MODEL))
    return out[:, None, :]


# speedup vs baseline: 2.2534x; 1.2136x over previous
import functools

import jax
import jax.numpy as jnp
from jax import lax
from jax.experimental import pallas as pl
from jax.experimental.pallas import tpu as pltpu
from jax.experimental.pallas import tpu_sc as plsc

_B = 16384
_N_UP = 1000000
_D_UP = 64
_D_SM = 32
_D_MODEL = 512

_NC = 2
_NS = 16
_NW = _NC * _NS
_BPW = _B // _NW
_CHUNK = 128
_NCHUNK = _BPW // _CHUNK

_RB = 8192


def _repack_body(tblT, out):
    tb = jnp.transpose(tblT[...].astype(jnp.bfloat16))
    t16 = lax.bitcast_convert_type(tb, jnp.uint16).astype(jnp.uint32)
    p = t16[: _RB // 2, :] | (t16[_RB // 2:, :] << 16)
    out_u = jnp.concatenate([p[: _RB // 4, :], p[_RB // 4:, :]], axis=-1)
    out[...] = lax.bitcast_convert_type(out_u, jnp.float32)


def _repack(tblT):
    grid = (pl.cdiv(_N_UP, _RB),)
    return pl.pallas_call(
        _repack_body,
        grid=grid,
        in_specs=[pl.BlockSpec((_D_UP, _RB), lambda i: (0, i))],
        out_specs=pl.BlockSpec((_RB // 4, 2 * _D_UP), lambda i: (i, 0)),
        out_shape=jax.ShapeDtypeStruct(
            (pl.cdiv(_N_UP, _RB) * (_RB // 4), 2 * _D_UP), jnp.float32),
        compiler_params=pltpu.CompilerParams(
            dimension_semantics=("parallel",),
        ),
    )(tblT)


def _sc_gather_body(jh, R, out, idx_v, rows_v, sem):
    wid = lax.axis_index("s") * _NC + lax.axis_index("c")
    base = wid * _BPW

    pltpu.sync_copy(jh.at[pl.ds(base, _BPW)], idx_v)
    copies = []
    for j in range(_NCHUNK):
        copies.append(pltpu.async_copy(
            R.at[idx_v.at[pl.ds(j * _CHUNK, _CHUNK)]],
            rows_v.at[pl.ds(j * _CHUNK, _CHUNK)], sem))
    for c in copies:
        c.wait()
    pltpu.sync_copy(rows_v, out.at[pl.ds(base, _BPW)])


@functools.cache
def _sc_gather():
    return pl.kernel(
        _sc_gather_body,
        out_type=jax.ShapeDtypeStruct((_B, 2 * _D_UP), jnp.float32),
        mesh=plsc.VectorSubcoreMesh(core_axis_name="c", subcore_axis_name="s",
                                    num_cores=_NC, num_subcores=_NS),
        scratch_types=[
            pltpu.VMEM((_BPW,), jnp.int32),
            pltpu.VMEM((_BPW, 2 * _D_UP), jnp.float32),
            pltpu.SemaphoreType.DMA,
        ],
    )


_BLK = 2048


def _onehot_part(idx_ref, tab_ref, w_slice, n_rows):
    idx = idx_ref[0]
    iota = lax.broadcasted_iota(jnp.int32, (n_rows, _BLK), 0)
    oh = jnp.equal(iota, idx).astype(jnp.float32)
    m = jnp.dot(tab_ref[...], w_slice, preferred_element_type=jnp.float32)
    return lax.dot_general(oh, m, (((0,), (0,)), ((), ())),
                           preferred_element_type=jnp.float32)


def _mlp_body(g_up, rem, pf, dt, lk, pf_tab, dt_tab, lk_tab,
              w1dd, W1, b1, gamma, beta, W2, b2, out):
    g_u = lax.bitcast_convert_type(g_up[...], jnp.uint32)
    f_lo = lax.bitcast_convert_type(g_u << 16, jnp.float32)
    f_hi = lax.bitcast_convert_type(g_u & jnp.uint32(0xFFFF0000), jnp.float32)
    u2 = lax.broadcasted_iota(jnp.int32, (_BLK, 2 * _D_UP), 1) // _D_UP
    m_lo = jnp.equal(rem[...], 2 * u2).astype(jnp.float32)
    m_hi = jnp.equal(rem[...], 2 * u2 + 1).astype(jnp.float32)
    e = f_lo * m_lo + f_hi * m_hi
    h = jnp.dot(e, w1dd[...], preferred_element_type=jnp.float32)
    h = h + _onehot_part(pf, pf_tab, W1[64:96, :], 10)
    h = h + _onehot_part(dt, dt_tab, W1[96:128, :], 12)
    h = h + _onehot_part(lk, lk_tab, W1[128:160, :], 10)
    h = h + b1[...]
    mean = jnp.mean(h, axis=-1, keepdims=True)
    c = h - mean
    var = jnp.mean(c * c, axis=-1, keepdims=True)
    h = c * lax.rsqrt(var + 1e-5) * gamma[...] + beta[...]
    h = jnp.maximum(h, 0.0)
    out[...] = jnp.dot(h, W2[...], preferred_element_type=jnp.float32) + b2[...]


def _mlp(g_up, rem, pf, dt, lk, pf_tab, dt_tab, lk_tab,
         w1dd, W1, b1, gamma, beta, W2, b2):
    grid = (_B // _BLK,)
    return pl.pallas_call(
        _mlp_body,
        grid=grid,
        in_specs=[
            pl.BlockSpec((_BLK, 2 * _D_UP), lambda i: (i, 0)),
            pl.BlockSpec((_BLK, 1), lambda i: (i, 0)),
            pl.BlockSpec((1, 1, _BLK), lambda i: (i, 0, 0)),
            pl.BlockSpec((1, 1, _BLK), lambda i: (i, 0, 0)),
            pl.BlockSpec((1, 1, _BLK), lambda i: (i, 0, 0)),
            pl.BlockSpec((10, _D_SM), lambda i: (0, 0)),
            pl.BlockSpec((12, _D_SM), lambda i: (0, 0)),
            pl.BlockSpec((10, _D_SM), lambda i: (0, 0)),
            pl.BlockSpec((2 * _D_UP, _D_MODEL), lambda i: (0, 0)),
            pl.BlockSpec((160, _D_MODEL), lambda i: (0, 0)),
            pl.BlockSpec((1, _D_MODEL), lambda i: (0, 0)),
            pl.BlockSpec((1, _D_MODEL), lambda i: (0, 0)),
            pl.BlockSpec((1, _D_MODEL), lambda i: (0, 0)),
            pl.BlockSpec((_D_MODEL, _D_MODEL), lambda i: (0, 0)),
            pl.BlockSpec((1, _D_MODEL), lambda i: (0, 0)),
        ],
        out_specs=pl.BlockSpec((_BLK, _D_MODEL), lambda i: (i, 0)),
        out_shape=jax.ShapeDtypeStruct((_B, _D_MODEL), jnp.float32),
        compiler_params=pltpu.CompilerParams(
            dimension_semantics=("arbitrary",),
        ),
    )(g_up, rem, pf, dt, lk, pf_tab, dt_tab, lk_tab,
      w1dd, W1, b1, gamma, beta, W2, b2)


def kernel(uploader, platform, date, likes, uploader_table, platform_table,
           date_table, likes_table, W1, b1, gamma, beta, W2, b2):
    nb = _B // _BLK
    upl = uploader.astype(jnp.int32)
    R = _repack(uploader_table.T)
    w = upl % _RB
    eta = w // (_RB // 2)
    r = w % (_RB // 2)
    v = r // (_RB // 4)
    q = (upl // _RB) * (_RB // 4) + (r % (_RB // 4))
    rem = 2 * v + eta
    g_up = _sc_gather()(q, R)

    w1dd = jnp.tile(W1[0:_D_UP, :], (2, 1))
    out = _mlp(g_up, rem.reshape(_B, 1),
               platform.astype(jnp.int32).reshape(nb, 1, _BLK),
               date.astype(jnp.int32).reshape(nb, 1, _BLK),
               likes.astype(jnp.int32).reshape(nb, 1, _BLK),
               platform_table, date_table, likes_table,
               w1dd,
               W1, b1.reshape(1, _D_MODEL), gamma.reshape(1, _D_MODEL),
               beta.reshape(1, _D_MODEL), W2, b2.reshape(1, _D_MODEL))
    return out[:, None, :]


# speedup vs baseline: 2.5798x; 1.1449x over previous
import functools

import jax
import jax.numpy as jnp
from jax import lax
from jax.experimental import pallas as pl
from jax.experimental.pallas import tpu as pltpu
from jax.experimental.pallas import tpu_sc as plsc

_B = 16384
_N_UP = 1000000
_D_UP = 64
_D_SM = 32
_D_MODEL = 512

_NC = 2
_NS = 16
_NW = _NC * _NS
_BPW = _B // _NW
_CHUNK = 128
_NCHUNK = _BPW // _CHUNK

_RB = 16384


def _repack_body(tblT, out):
    tb = jnp.transpose(tblT[...].astype(jnp.bfloat16))
    t16 = lax.bitcast_convert_type(tb, jnp.uint16).astype(jnp.uint32)
    p = t16[: _RB // 2, :] | (t16[_RB // 2:, :] << 16)
    out_u = jnp.concatenate([p[: _RB // 4, :], p[_RB // 4:, :]], axis=-1)
    out[...] = lax.bitcast_convert_type(out_u, jnp.float32)


def _repack(tblT):
    grid = (pl.cdiv(_N_UP, _RB),)
    return pl.pallas_call(
        _repack_body,
        grid=grid,
        in_specs=[pl.BlockSpec((_D_UP, _RB), lambda i: (0, i))],
        out_specs=pl.BlockSpec((_RB // 4, 2 * _D_UP), lambda i: (i, 0)),
        out_shape=jax.ShapeDtypeStruct(
            (pl.cdiv(_N_UP, _RB) * (_RB // 4), 2 * _D_UP), jnp.float32),
        compiler_params=pltpu.CompilerParams(
            dimension_semantics=("parallel",),
        ),
    )(tblT)


def _sc_gather_body(jh, R, out, idx_v, rows_v, sem):
    wid = lax.axis_index("s") * _NC + lax.axis_index("c")
    base = wid * _BPW

    pltpu.sync_copy(jh.at[pl.ds(base, _BPW)], idx_v)
    copies = []
    for j in range(_NCHUNK):
        copies.append(pltpu.async_copy(
            R.at[idx_v.at[pl.ds(j * _CHUNK, _CHUNK)]],
            rows_v.at[pl.ds(j * _CHUNK, _CHUNK)], sem))
    for c in copies:
        c.wait()
    pltpu.sync_copy(rows_v, out.at[pl.ds(base, _BPW)])


@functools.cache
def _sc_gather():
    return pl.kernel(
        _sc_gather_body,
        out_type=jax.ShapeDtypeStruct((_B, 2 * _D_UP), jnp.float32),
        mesh=plsc.VectorSubcoreMesh(core_axis_name="c", subcore_axis_name="s",
                                    num_cores=_NC, num_subcores=_NS),
        scratch_types=[
            pltpu.VMEM((_BPW,), jnp.int32),
            pltpu.VMEM((_BPW, 2 * _D_UP), jnp.float32),
            pltpu.SemaphoreType.DMA,
        ],
    )


_BLK = 2048


def _onehot_part(idx_ref, tab_ref, w_slice, n_rows):
    idx = idx_ref[0]
    iota = lax.broadcasted_iota(jnp.int32, (n_rows, _BLK), 0)
    oh = jnp.equal(iota, idx).astype(jnp.float32)
    m = jnp.dot(tab_ref[...], w_slice, preferred_element_type=jnp.float32)
    return lax.dot_general(oh, m, (((0,), (0,)), ((), ())),
                           preferred_element_type=jnp.float32)


def _mlp_body(g_up, rem, pf, dt, lk, pf_tab, dt_tab, lk_tab,
              w1dd, W1, b1, gamma, beta, W2, b2, out):
    g_u = lax.bitcast_convert_type(g_up[...], jnp.uint32)
    f_lo = lax.bitcast_convert_type(g_u << 16, jnp.float32)
    f_hi = lax.bitcast_convert_type(g_u & jnp.uint32(0xFFFF0000), jnp.float32)
    u2 = lax.broadcasted_iota(jnp.int32, (_BLK, 2 * _D_UP), 1) // _D_UP
    m_lo = jnp.equal(rem[...], 2 * u2).astype(jnp.float32)
    m_hi = jnp.equal(rem[...], 2 * u2 + 1).astype(jnp.float32)
    e = f_lo * m_lo + f_hi * m_hi
    h = jnp.dot(e, w1dd[...], preferred_element_type=jnp.float32)
    h = h + _onehot_part(pf, pf_tab, W1[64:96, :], 10)
    h = h + _onehot_part(dt, dt_tab, W1[96:128, :], 12)
    h = h + _onehot_part(lk, lk_tab, W1[128:160, :], 10)
    h = h + b1[...]
    mean = jnp.mean(h, axis=-1, keepdims=True)
    c = h - mean
    var = jnp.mean(c * c, axis=-1, keepdims=True)
    h = c * lax.rsqrt(var + 1e-5) * gamma[...] + beta[...]
    h = jnp.maximum(h, 0.0)
    out[...] = jnp.dot(h, W2[...], preferred_element_type=jnp.float32) + b2[...]


def _mlp(g_up, rem, pf, dt, lk, pf_tab, dt_tab, lk_tab,
         w1dd, W1, b1, gamma, beta, W2, b2):
    grid = (_B // _BLK,)
    return pl.pallas_call(
        _mlp_body,
        grid=grid,
        in_specs=[
            pl.BlockSpec((_BLK, 2 * _D_UP), lambda i: (i, 0)),
            pl.BlockSpec((_BLK, 1), lambda i: (i, 0)),
            pl.BlockSpec((1, 1, _BLK), lambda i: (i, 0, 0)),
            pl.BlockSpec((1, 1, _BLK), lambda i: (i, 0, 0)),
            pl.BlockSpec((1, 1, _BLK), lambda i: (i, 0, 0)),
            pl.BlockSpec((10, _D_SM), lambda i: (0, 0)),
            pl.BlockSpec((12, _D_SM), lambda i: (0, 0)),
            pl.BlockSpec((10, _D_SM), lambda i: (0, 0)),
            pl.BlockSpec((2 * _D_UP, _D_MODEL), lambda i: (0, 0)),
            pl.BlockSpec((160, _D_MODEL), lambda i: (0, 0)),
            pl.BlockSpec((1, _D_MODEL), lambda i: (0, 0)),
            pl.BlockSpec((1, _D_MODEL), lambda i: (0, 0)),
            pl.BlockSpec((1, _D_MODEL), lambda i: (0, 0)),
            pl.BlockSpec((_D_MODEL, _D_MODEL), lambda i: (0, 0)),
            pl.BlockSpec((1, _D_MODEL), lambda i: (0, 0)),
        ],
        out_specs=pl.BlockSpec((_BLK, _D_MODEL), lambda i: (i, 0)),
        out_shape=jax.ShapeDtypeStruct((_B, _D_MODEL), jnp.float32),
        compiler_params=pltpu.CompilerParams(
            dimension_semantics=("arbitrary",),
        ),
    )(g_up, rem, pf, dt, lk, pf_tab, dt_tab, lk_tab,
      w1dd, W1, b1, gamma, beta, W2, b2)


def kernel(uploader, platform, date, likes, uploader_table, platform_table,
           date_table, likes_table, W1, b1, gamma, beta, W2, b2):
    nb = _B // _BLK
    upl = uploader.astype(jnp.int32)
    R = _repack(uploader_table.T)
    w = upl % _RB
    eta = w // (_RB // 2)
    r = w % (_RB // 2)
    v = r // (_RB // 4)
    q = (upl // _RB) * (_RB // 4) + (r % (_RB // 4))
    rem = 2 * v + eta
    g_up = _sc_gather()(q, R)

    w1dd = jnp.tile(W1[0:_D_UP, :], (2, 1))
    out = _mlp(g_up, rem.reshape(_B, 1),
               platform.astype(jnp.int32).reshape(nb, 1, _BLK),
               date.astype(jnp.int32).reshape(nb, 1, _BLK),
               likes.astype(jnp.int32).reshape(nb, 1, _BLK),
               platform_table, date_table, likes_table,
               w1dd,
               W1, b1.reshape(1, _D_MODEL), gamma.reshape(1, _D_MODEL),
               beta.reshape(1, _D_MODEL), W2, b2.reshape(1, _D_MODEL))
    return out[:, None, :]


# speedup vs baseline: 2.7741x; 1.0753x over previous
import functools

import jax
import jax.numpy as jnp
from jax import lax
from jax.experimental import pallas as pl
from jax.experimental.pallas import tpu as pltpu
from jax.experimental.pallas import tpu_sc as plsc

_B = 16384
_N_UP = 1000000
_D_UP = 64
_D_SM = 32
_D_MODEL = 512

_NC = 2
_NS = 16
_NW = _NC * _NS
_BPW = _B // _NW
_CHUNK = 128
_NCHUNK = _BPW // _CHUNK

_RB = 32768


def _repack_body(tblT, out):
    tb = jnp.transpose(tblT[...].astype(jnp.bfloat16))
    t16 = lax.bitcast_convert_type(tb, jnp.uint16).astype(jnp.uint32)
    p = t16[: _RB // 2, :] | (t16[_RB // 2:, :] << 16)
    out_u = jnp.concatenate([p[: _RB // 4, :], p[_RB // 4:, :]], axis=-1)
    out[...] = lax.bitcast_convert_type(out_u, jnp.float32)


def _repack(tblT):
    grid = (pl.cdiv(_N_UP, _RB),)
    return pl.pallas_call(
        _repack_body,
        grid=grid,
        in_specs=[pl.BlockSpec((_D_UP, _RB), lambda i: (0, i))],
        out_specs=pl.BlockSpec((_RB // 4, 2 * _D_UP), lambda i: (i, 0)),
        out_shape=jax.ShapeDtypeStruct(
            (pl.cdiv(_N_UP, _RB) * (_RB // 4), 2 * _D_UP), jnp.float32),
        compiler_params=pltpu.CompilerParams(
            dimension_semantics=("parallel",),
        ),
    )(tblT)


def _sc_gather_body(jh, R, out, idx_v, rows_v, sem):
    wid = lax.axis_index("s") * _NC + lax.axis_index("c")
    base = wid * _BPW

    pltpu.sync_copy(jh.at[pl.ds(base, _BPW)], idx_v)
    copies = []
    for j in range(_NCHUNK):
        copies.append(pltpu.async_copy(
            R.at[idx_v.at[pl.ds(j * _CHUNK, _CHUNK)]],
            rows_v.at[pl.ds(j * _CHUNK, _CHUNK)], sem))
    for c in copies:
        c.wait()
    pltpu.sync_copy(rows_v, out.at[pl.ds(base, _BPW)])


@functools.cache
def _sc_gather():
    return pl.kernel(
        _sc_gather_body,
        out_type=jax.ShapeDtypeStruct((_B, 2 * _D_UP), jnp.float32),
        mesh=plsc.VectorSubcoreMesh(core_axis_name="c", subcore_axis_name="s",
                                    num_cores=_NC, num_subcores=_NS),
        scratch_types=[
            pltpu.VMEM((_BPW,), jnp.int32),
            pltpu.VMEM((_BPW, 2 * _D_UP), jnp.float32),
            pltpu.SemaphoreType.DMA,
        ],
    )


_BLK = 2048


def _onehot_part(idx_ref, tab_ref, w_slice, n_rows):
    idx = idx_ref[0]
    iota = lax.broadcasted_iota(jnp.int32, (n_rows, _BLK), 0)
    oh = jnp.equal(iota, idx).astype(jnp.float32)
    m = jnp.dot(tab_ref[...], w_slice, preferred_element_type=jnp.float32)
    return lax.dot_general(oh, m, (((0,), (0,)), ((), ())),
                           preferred_element_type=jnp.float32)


def _mlp_body(g_up, rem, pf, dt, lk, pf_tab, dt_tab, lk_tab,
              w1dd, W1, b1, gamma, beta, W2, b2, out):
    g_u = lax.bitcast_convert_type(g_up[...], jnp.uint32)
    f_lo = lax.bitcast_convert_type(g_u << 16, jnp.float32)
    f_hi = lax.bitcast_convert_type(g_u & jnp.uint32(0xFFFF0000), jnp.float32)
    u2 = lax.broadcasted_iota(jnp.int32, (_BLK, 2 * _D_UP), 1) // _D_UP
    m_lo = jnp.equal(rem[...], 2 * u2).astype(jnp.float32)
    m_hi = jnp.equal(rem[...], 2 * u2 + 1).astype(jnp.float32)
    e = f_lo * m_lo + f_hi * m_hi
    h = jnp.dot(e, w1dd[...], preferred_element_type=jnp.float32)
    h = h + _onehot_part(pf, pf_tab, W1[64:96, :], 10)
    h = h + _onehot_part(dt, dt_tab, W1[96:128, :], 12)
    h = h + _onehot_part(lk, lk_tab, W1[128:160, :], 10)
    h = h + b1[...]
    mean = jnp.mean(h, axis=-1, keepdims=True)
    c = h - mean
    var = jnp.mean(c * c, axis=-1, keepdims=True)
    h = c * lax.rsqrt(var + 1e-5) * gamma[...] + beta[...]
    h = jnp.maximum(h, 0.0)
    out[...] = jnp.dot(h, W2[...], preferred_element_type=jnp.float32) + b2[...]


def _mlp(g_up, rem, pf, dt, lk, pf_tab, dt_tab, lk_tab,
         w1dd, W1, b1, gamma, beta, W2, b2):
    grid = (_B // _BLK,)
    return pl.pallas_call(
        _mlp_body,
        grid=grid,
        in_specs=[
            pl.BlockSpec((_BLK, 2 * _D_UP), lambda i: (i, 0)),
            pl.BlockSpec((_BLK, 1), lambda i: (i, 0)),
            pl.BlockSpec((1, 1, _BLK), lambda i: (i, 0, 0)),
            pl.BlockSpec((1, 1, _BLK), lambda i: (i, 0, 0)),
            pl.BlockSpec((1, 1, _BLK), lambda i: (i, 0, 0)),
            pl.BlockSpec((10, _D_SM), lambda i: (0, 0)),
            pl.BlockSpec((12, _D_SM), lambda i: (0, 0)),
            pl.BlockSpec((10, _D_SM), lambda i: (0, 0)),
            pl.BlockSpec((2 * _D_UP, _D_MODEL), lambda i: (0, 0)),
            pl.BlockSpec((160, _D_MODEL), lambda i: (0, 0)),
            pl.BlockSpec((1, _D_MODEL), lambda i: (0, 0)),
            pl.BlockSpec((1, _D_MODEL), lambda i: (0, 0)),
            pl.BlockSpec((1, _D_MODEL), lambda i: (0, 0)),
            pl.BlockSpec((_D_MODEL, _D_MODEL), lambda i: (0, 0)),
            pl.BlockSpec((1, _D_MODEL), lambda i: (0, 0)),
        ],
        out_specs=pl.BlockSpec((_BLK, _D_MODEL), lambda i: (i, 0)),
        out_shape=jax.ShapeDtypeStruct((_B, _D_MODEL), jnp.float32),
        compiler_params=pltpu.CompilerParams(
            dimension_semantics=("arbitrary",),
        ),
    )(g_up, rem, pf, dt, lk, pf_tab, dt_tab, lk_tab,
      w1dd, W1, b1, gamma, beta, W2, b2)


def kernel(uploader, platform, date, likes, uploader_table, platform_table,
           date_table, likes_table, W1, b1, gamma, beta, W2, b2):
    nb = _B // _BLK
    upl = uploader.astype(jnp.int32)
    R = _repack(uploader_table.T)
    w = upl % _RB
    eta = w // (_RB // 2)
    r = w % (_RB // 2)
    v = r // (_RB // 4)
    q = (upl // _RB) * (_RB // 4) + (r % (_RB // 4))
    rem = 2 * v + eta
    g_up = _sc_gather()(q, R)

    w1dd = jnp.tile(W1[0:_D_UP, :], (2, 1))
    out = _mlp(g_up, rem.reshape(_B, 1),
               platform.astype(jnp.int32).reshape(nb, 1, _BLK),
               date.astype(jnp.int32).reshape(nb, 1, _BLK),
               likes.astype(jnp.int32).reshape(nb, 1, _BLK),
               platform_table, date_table, likes_table,
               w1dd,
               W1, b1.reshape(1, _D_MODEL), gamma.reshape(1, _D_MODEL),
               beta.reshape(1, _D_MODEL), W2, b2.reshape(1, _D_MODEL))
    return out[:, None, :]
